# Initial kernel scaffold; baseline (speedup 1.0000x reference)
import jax
import jax.numpy as jnp
from jax.experimental import pallas as pl


def kernel(h, edge_index, etypes, weight, ln_gamma, ln_beta):
    raise NotImplementedError("write your pallas kernel here")


# speedup vs baseline: 4.3482x; 4.3482x over previous
import functools

import jax
import jax.numpy as jnp
import numpy as np
from jax import lax
from jax.experimental import pallas as pl
from jax.experimental.pallas import tpu as pltpu
from jax.experimental.pallas import tpu_sc as plsc

N_NODES = 10000
H_DIM = 128
NUM_R = 200
HALF = H_DIM // 2

NW = 32
CHUNK = 80
N_PAD = 10240
ROWS_PER_TILE = N_PAD // 16
DEG0 = N_PAD
DEG_ROWS = 128
ZROWS = 16


def _sc_aggregate(h_perm, src, dst, etypes, wde):
    E = src.shape[0]
    e_per_w = E // NW
    n_chunks = e_per_w // CHUNK
    zrows_per_tile = (N_PAD + DEG_ROWS) // 16
    mesh = plsc.VectorSubcoreMesh(core_axis_name="c", subcore_axis_name="s")

    @functools.partial(
        pl.kernel,
        mesh=mesh,
        out_type=[
            jax.ShapeDtypeStruct((2, N_PAD, H_DIM), jnp.float32),
            jax.ShapeDtypeStruct((2, N_PAD, H_DIM), jnp.float32),
        ],
        scratch_types=[
            pltpu.VMEM((CHUNK, H_DIM), jnp.float32),
            pltpu.VMEM((CHUNK, 2 * H_DIM), jnp.float32),
            pltpu.VMEM((N_PAD // H_DIM, H_DIM), jnp.float32),
            pltpu.VMEM((ZROWS, H_DIM), jnp.float32),
            pltpu.VMEM((8, H_DIM), jnp.float32),
            pltpu.VMEM((CHUNK,), jnp.int32),
            pltpu.VMEM((CHUNK,), jnp.int32),
            pltpu.VMEM((CHUNK,), jnp.int32),
            pltpu.VMEM((CHUNK,), jnp.int32),
            pltpu.VMEM((16, 16), jnp.float32),
            pltpu.VMEM_SHARED((N_PAD + DEG_ROWS, H_DIM), jnp.float32),
            pltpu.SemaphoreType.DMA,
            pltpu.SemaphoreType.DMA,
        ],
    )
    def k(hh, srch, dsth, eth, wdeh, didxh, out, degout, hbuf, wbuf, degv,
          zbuf, degrd, srcv, dstv, etv, didx, ohtab, agg, sem, sem2):
        core = lax.axis_index("c")
        sub = lax.axis_index("s")
        wid = sub * 2 + core

        zvec = jnp.zeros((16,), jnp.float32)
        lanes = lax.iota(jnp.int32, 16)

        def zrow(r, _):
            for j in range(H_DIM // 16):
                zbuf[r, pl.ds(j * 16, 16)] = zvec
            return 0

        lax.fori_loop(0, ZROWS, zrow, 0)

        def zdeg(r, _):
            for j in range(H_DIM // 16):
                degv[r, pl.ds(j * 16, 16)] = zvec
            return 0

        lax.fori_loop(0, N_PAD // H_DIM, zdeg, 0)

        def zacc(r, _):
            pltpu.sync_copy(
                zbuf, agg.at[pl.ds(sub * zrows_per_tile + r * ZROWS, ZROWS)])
            return 0

        lax.fori_loop(0, zrows_per_tile // ZROWS, zacc, 0)
        pltpu.sync_copy(
            zbuf.at[pl.ds(0, 8)],
            agg.at[pl.ds(sub * zrows_per_tile
                         + (zrows_per_tile // ZROWS) * ZROWS, 8)])

        pltpu.sync_copy(didxh, didx)
        for j in range(16):
            ohtab[j, pl.ds(0, 16)] = jnp.where(lanes == j, 1.0, 0.0)

        plsc.subcore_barrier()

        def chunk_body(c, _):
            base = wid * e_per_w + c * CHUNK
            pltpu.sync_copy(srch.at[pl.ds(base, CHUNK)], srcv)
            pltpu.sync_copy(dsth.at[pl.ds(base, CHUNK)], dstv)
            pltpu.sync_copy(eth.at[pl.ds(base, CHUNK)], etv)
            cp1 = pltpu.async_copy(hh.at[srcv], hbuf, sem)
            cp2 = pltpu.async_copy(wdeh.at[etv], wbuf, sem2)
            cp1.wait()
            cp2.wait()

            def edge16(i16, _):
                dstvec = dstv[pl.ds(i16 * 16, 16)]
                drowv = dstvec >> 7
                dcolv = ((dstvec >> 4) & 7) * 16
                dlanev = dstvec & 15
                for j in range(16):
                    i = i16 * 16 + j
                    drow = drowv[j]
                    dcol = dcolv[j]
                    oh = ohtab[dlanev[j], pl.ds(0, 16)]
                    degv[drow, pl.ds(dcol, 16)] = (
                        degv[drow, pl.ds(dcol, 16)] + oh)
                    for g in range(HALF // 16):
                        he = hbuf[i, pl.ds(g * 16, 16)]
                        ho = hbuf[i, pl.ds(HALF + g * 16, 16)]
                        w00 = wbuf[i, pl.ds(g * 16, 16)]
                        w10 = wbuf[i, pl.ds(HALF + g * 16, 16)]
                        w01 = wbuf[i, pl.ds(2 * HALF + g * 16, 16)]
                        w11 = wbuf[i, pl.ds(3 * HALF + g * 16, 16)]
                        hbuf[i, pl.ds(g * 16, 16)] = he * w00 + ho * w10
                        hbuf[i, pl.ds(HALF + g * 16, 16)] = he * w01 + ho * w11
                return 0

            lax.fori_loop(0, CHUNK // 16, edge16, 0)
            pltpu.sync_copy(hbuf, agg.at[dstv], add=True)
            return 0

        lax.fori_loop(0, n_chunks, chunk_body, 0)

        pltpu.sync_copy(degv, agg.at[didx], add=True)
        plsc.subcore_barrier()

        pltpu.sync_copy(agg.at[pl.ds(DEG0 + sub * 8, 8)], degrd)

        def outk(kk, _):
            q = kk // 8
            c16 = (kk - q * 8) * 16
            dvec = degrd[q, pl.ds(c16, 16)]
            for j in range(16):
                sv = jnp.full((16,), dvec[j], jnp.float32)
                for cc in range(H_DIM // 16):
                    zbuf[j, pl.ds(cc * 16, 16)] = sv
            r0 = sub * ROWS_PER_TILE + kk * 16
            pltpu.sync_copy(zbuf, degout.at[core, pl.ds(r0, 16)])
            pltpu.sync_copy(agg.at[pl.ds(r0, 16)],
                            out.at[core, pl.ds(r0, 16)])
            return 0

        lax.fori_loop(0, ROWS_PER_TILE // 16, outk, 0)

    r = np.arange(CHUNK, dtype=np.int32)
    didx_host = jnp.asarray(DEG0 + (r // 5) * 8 + (r % 5), dtype=jnp.int32)
    return k(h_perm, src, dst, etypes, wde, didx_host)


def _tc_finish(parts, degs, gamma, beta, perm_mat):
    n_blk = 2000
    grid = (N_NODES // n_blk,)

    def body(parts_ref, deg_ref, g_ref, b_ref, p_ref, o_ref):
        x = parts_ref[0] + parts_ref[1]
        deg = deg_ref[0] + deg_ref[1]
        x = x * (1.0 / jnp.maximum(deg, 1.0))
        mean = jnp.mean(x, axis=1, keepdims=True)
        xc = x - mean
        var = jnp.mean(xc * xc, axis=1, keepdims=True)
        y = xc * lax.rsqrt(var + 1e-5)
        y = jnp.dot(y, p_ref[...], preferred_element_type=jnp.float32)
        o_ref[...] = y * g_ref[...] + b_ref[...]

    return pl.pallas_call(
        body,
        grid=grid,
        in_specs=[
            pl.BlockSpec((2, n_blk, H_DIM), lambda i: (0, i, 0)),
            pl.BlockSpec((2, n_blk, H_DIM), lambda i: (0, i, 0)),
            pl.BlockSpec((1, H_DIM), lambda i: (0, 0)),
            pl.BlockSpec((1, H_DIM), lambda i: (0, 0)),
            pl.BlockSpec((H_DIM, H_DIM), lambda i: (0, 0)),
        ],
        out_specs=pl.BlockSpec((n_blk, H_DIM), lambda i: (i, 0)),
        out_shape=jax.ShapeDtypeStruct((N_NODES, H_DIM), jnp.float32),
    )(parts, degs, gamma, beta, perm_mat)


def kernel(h, edge_index, etypes, weight, ln_gamma, ln_beta):
    src = edge_index[0].astype(jnp.int32)
    dst = edge_index[1].astype(jnp.int32)
    et = etypes.astype(jnp.int32)

    h_perm = jnp.concatenate([h[:, 0::2], h[:, 1::2]], axis=1)

    wt = weight.reshape(NUM_R, HALF, 2, 2)
    wde = jnp.concatenate(
        [wt[:, :, 0, 0], wt[:, :, 1, 0], wt[:, :, 0, 1], wt[:, :, 1, 1]],
        axis=1,
    )

    parts, degs = _sc_aggregate(h_perm, src, dst, et, wde)
    parts = parts[:, :N_NODES]
    degs = degs[:, :N_NODES]

    pm = np.zeros((H_DIM, H_DIM), dtype=np.float32)
    for j in range(HALF):
        pm[j, 2 * j] = 1.0
        pm[HALF + j, 2 * j + 1] = 1.0
    perm_mat = jnp.asarray(pm)

    return _tc_finish(parts, degs, ln_gamma.reshape(1, H_DIM),
                      ln_beta.reshape(1, H_DIM), perm_mat)


# speedup vs baseline: 6.7270x; 1.5471x over previous
import functools

import jax
import jax.numpy as jnp
import numpy as np
from jax import lax
from jax.experimental import pallas as pl
from jax.experimental.pallas import tpu as pltpu
from jax.experimental.pallas import tpu_sc as plsc

N_NODES = 10000
H_DIM = 128
NUM_R = 200
HALF = H_DIM // 2

NW = 32
CHUNK = 80
N_PAD = 10240
ROWS_PER_TILE = N_PAD // 16
DEG_ROWS = 128


def _sc_aggregate(h_perm, src, dst, etypes, w32, didx_host):
    E = src.shape[0]
    e_per_w = E // NW
    n_chunks = e_per_w // CHUNK
    mesh = plsc.VectorSubcoreMesh(core_axis_name="c", subcore_axis_name="s")

    @functools.partial(
        pl.kernel,
        mesh=mesh,
        compiler_params=pltpu.CompilerParams(needs_layout_passes=False),
        out_type=[
            jax.ShapeDtypeStruct((2, N_PAD, H_DIM), jnp.float32),
            jax.ShapeDtypeStruct((2, N_PAD, H_DIM), jnp.float32),
        ],
        scratch_types=[
            pltpu.VMEM((2, CHUNK, H_DIM), jnp.float32),
            pltpu.VMEM((2, CHUNK, H_DIM), jnp.int32),
            pltpu.VMEM((CHUNK,), jnp.int32),
            pltpu.VMEM((CHUNK,), jnp.int32),
            pltpu.VMEM((CHUNK,), jnp.int32),
            pltpu.VMEM((CHUNK,), jnp.int32),
            pltpu.VMEM((CHUNK,), jnp.int32),
            pltpu.VMEM((CHUNK,), jnp.int32),
            pltpu.VMEM((CHUNK // 2, H_DIM), jnp.int32),
            pltpu.VMEM((4, H_DIM), jnp.int32),
            pltpu.VMEM_SHARED((N_PAD, H_DIM), jnp.float32),
            pltpu.VMEM_SHARED((DEG_ROWS, H_DIM), jnp.float32),
            pltpu.SemaphoreType.DMA,
            pltpu.SemaphoreType.DMA,
            pltpu.SemaphoreType.DMA,
            pltpu.SemaphoreType.DMA,
            pltpu.SemaphoreType.DMA,
            pltpu.SemaphoreType.DMA,
        ],
    )
    def k(hh, srch, dsth, eth, wdeh, didxh, out, degout, hbr, wbr,
          srcv0, srcv1, dstv0, dstv1, etv0, etv1, degv, ohtab, agg, dagg,
          gh0, gh1, gw0, gw1, sc0, sc1):
        core = lax.axis_index("c")
        sub = lax.axis_index("s")
        wid = sub * 2 + core
        ghs = (gh0, gh1)
        gws = (gw0, gw1)
        scs = (sc0, sc1)
        srcvs = (srcv0, srcv1)
        dstvs = (dstv0, dstv1)
        etvs = (etv0, etv1)
        hb0 = hbr.at[0]

        zvec = jnp.zeros((16,), jnp.float32)
        zivec = jnp.zeros((16,), jnp.int32)
        lanes = lax.iota(jnp.int32, 16)

        def zrow(r, _):
            for j in range(H_DIM // 16):
                hbr[0, r, pl.ds(j * 16, 16)] = zvec
            return 0

        lax.fori_loop(0, CHUNK, zrow, 0)

        def zdeg(r, _):
            for j in range(H_DIM // 16):
                degv[r, pl.ds(j * 16, 16)] = zivec
            return 0

        lax.fori_loop(0, CHUNK // 2, zdeg, 0)

        def zacc(r, _):
            pltpu.sync_copy(
                hb0, agg.at[pl.ds(sub * ROWS_PER_TILE + r * CHUNK, CHUNK)])
            return 0

        lax.fori_loop(0, ROWS_PER_TILE // CHUNK, zacc, 0)
        pltpu.sync_copy(hb0.at[pl.ds(0, 8)], dagg.at[pl.ds(sub * 8, 8)])

        for kk in range(32):
            ohtab[kk // 8, pl.ds((kk % 8) * 16, 16)] = jnp.where(
                lanes == (kk & 15), 1 << (16 * (kk >> 4)), 0)

        plsc.subcore_barrier()

        def load_idx(c, b):
            base = wid * e_per_w + c * CHUNK
            pltpu.sync_copy(srch.at[pl.ds(base, CHUNK)], srcvs[b])
            pltpu.sync_copy(dsth.at[pl.ds(base, CHUNK)], dstvs[b])
            pltpu.sync_copy(eth.at[pl.ds(base, CHUNK)], etvs[b])

        def start_gathers(b):
            return (pltpu.async_copy(hh.at[srcvs[b]], hbr.at[b], ghs[b]),
                    pltpu.async_copy(wdeh.at[etvs[b]], wbr.at[b], gws[b]))

        def wait_gathers(b):
            pltpu.make_async_copy(hh.at[srcvs[b]], hbr.at[b], ghs[b]).wait()
            pltpu.make_async_copy(wdeh.at[etvs[b]], wbr.at[b], gws[b]).wait()

        def compute(b):
            def edge16(i16, _):
                dstvec = dstvs[b][pl.ds(i16 * 16, 16)]
                wrowv = dstvec >> 8
                wcolv = ((dstvec >> 5) & 7) * 16
                ohidx = (dstvec & 1) * 16 + ((dstvec >> 1) & 15)
                ohrowv = ohidx >> 3
                ohcolv = (ohidx & 7) * 16
                for j in range(16):
                    i = i16 * 16 + j
                    oh = ohtab[ohrowv[j], pl.ds(ohcolv[j], 16)]
                    degv[wrowv[j], pl.ds(wcolv[j], 16)] = (
                        degv[wrowv[j], pl.ds(wcolv[j], 16)] + oh)
                    for g in range(HALF // 16):
                        he = hbr[b, i, pl.ds(g * 16, 16)]
                        ho = hbr[b, i, pl.ds(HALF + g * 16, 16)]
                        wv0 = plsc.bitcast(
                            wbr[b, i, pl.ds(g * 32, 16)], jnp.bfloat16)
                        wv1 = plsc.bitcast(
                            wbr[b, i, pl.ds(g * 32 + 16, 16)], jnp.bfloat16)
                        w00, w10 = plsc.unpack(
                            wv0, format=plsc.PackFormat.INTERLEAVED)
                        w01, w11 = plsc.unpack(
                            wv1, format=plsc.PackFormat.INTERLEAVED)
                        hbr[b, i, pl.ds(g * 16, 16)] = he * w00 + ho * w10
                        hbr[b, i, pl.ds(HALF + g * 16, 16)] = (
                            he * w01 + ho * w11)
                return 0

            lax.fori_loop(0, CHUNK // 16, edge16, 0)

        def start_scatter(b):
            return pltpu.async_copy(hbr.at[b], agg.at[dstvs[b]], scs[b],
                                    add=True)

        def wait_scatter(b):
            pltpu.make_async_copy(hbr.at[b], agg.at[dstvs[b]],
                                  scs[b]).wait()

        load_idx(0, 0)
        start_gathers(0)
        load_idx(1, 1)
        start_gathers(1)
        wait_gathers(0)
        compute(0)
        start_scatter(0)

        def step(c, b):
            wait_gathers(b)
            wait_scatter(1 - b)
            load_idx(c + 1, 1 - b)
            start_gathers(1 - b)
            compute(b)
            start_scatter(b)

        def pair(c2, _):
            c = 2 * c2 + 1
            step(c, 1)
            step(c + 1, 0)
            return 0

        lax.fori_loop(0, (n_chunks - 3) // 2, pair, 0)

        wait_gathers(1)
        wait_scatter(0)
        load_idx(n_chunks - 1, 0)
        start_gathers(0)
        compute(1)
        start_scatter(1)
        wait_gathers(0)
        wait_scatter(1)
        compute(0)
        start_scatter(0)
        wait_scatter(0)

        def dconv(r, _):
            for hrow in range(2):
                for t in range(HALF // 16):
                    v = degv[r, pl.ds(hrow * HALF + t * 16, 16)]
                    hbr[0, 2 * r + hrow, pl.ds(t * 16, 16)] = (
                        (v & 0xFFFF).astype(jnp.float32))
                    hbr[0, 2 * r + hrow, pl.ds(HALF + t * 16, 16)] = (
                        (v >> 16).astype(jnp.float32))
            return 0

        lax.fori_loop(0, CHUNK // 2, dconv, 0)
        pltpu.sync_copy(didxh, etvs[0])
        pltpu.sync_copy(hb0, dagg.at[etvs[0]], add=True)
        plsc.subcore_barrier()

        pltpu.sync_copy(dagg.at[pl.ds(sub * 8, 8)], hb0.at[pl.ds(0, 8)])

        def outk(kk2, _):
            q = kk2 // 4
            e0 = (kk2 - q * 4) * 16
            evec = hbr[0, q, pl.ds(e0, 16)]
            ovec = hbr[0, q, pl.ds(HALF + e0, 16)]
            for j in range(32):
                val = evec[j // 2] if j % 2 == 0 else ovec[j // 2]
                sv = jnp.full((16,), val, jnp.float32)
                for cc in range(H_DIM // 16):
                    hbr[0, 16 + j, pl.ds(cc * 16, 16)] = sv
            r0 = sub * ROWS_PER_TILE + kk2 * 32
            pltpu.sync_copy(hb0.at[pl.ds(16, 32)],
                            degout.at[core, pl.ds(r0, 32)])
            pltpu.sync_copy(agg.at[pl.ds(r0, 32)],
                            out.at[core, pl.ds(r0, 32)])
            return 0

        lax.fori_loop(0, ROWS_PER_TILE // 32, outk, 0)

    return k(h_perm, src, dst, etypes, w32, didx_host)


def _tc_finish(parts, degs, gamma, beta, perm_mat):
    n_blk = 2000
    grid = (N_NODES // n_blk,)

    def body(parts_ref, deg_ref, g_ref, b_ref, p_ref, o_ref):
        x = parts_ref[0] + parts_ref[1]
        deg = deg_ref[0] + deg_ref[1]
        x = x * (1.0 / jnp.maximum(deg, 1.0))
        mean = jnp.mean(x, axis=1, keepdims=True)
        xc = x - mean
        var = jnp.mean(xc * xc, axis=1, keepdims=True)
        y = xc * lax.rsqrt(var + 1e-5)
        y = jnp.dot(y, p_ref[...], preferred_element_type=jnp.float32)
        o_ref[...] = y * g_ref[...] + b_ref[...]

    return pl.pallas_call(
        body,
        grid=grid,
        in_specs=[
            pl.BlockSpec((2, n_blk, H_DIM), lambda i: (0, i, 0)),
            pl.BlockSpec((2, n_blk, H_DIM), lambda i: (0, i, 0)),
            pl.BlockSpec((1, H_DIM), lambda i: (0, 0)),
            pl.BlockSpec((1, H_DIM), lambda i: (0, 0)),
            pl.BlockSpec((H_DIM, H_DIM), lambda i: (0, 0)),
        ],
        out_specs=pl.BlockSpec((n_blk, H_DIM), lambda i: (i, 0)),
        out_shape=jax.ShapeDtypeStruct((N_NODES, H_DIM), jnp.float32),
    )(parts, degs, gamma, beta, perm_mat)


def kernel(h, edge_index, etypes, weight, ln_gamma, ln_beta):
    src = edge_index[0].astype(jnp.int32)
    dst = edge_index[1].astype(jnp.int32)
    et = etypes.astype(jnp.int32)

    h_perm = jnp.concatenate([h[:, 0::2], h[:, 1::2]], axis=1)

    wt = weight.reshape(NUM_R, HALF, 2, 2).astype(jnp.bfloat16)
    ilv_a = jnp.stack([wt[:, :, 0, 0], wt[:, :, 1, 0]], axis=-1)
    ilv_b = jnp.stack([wt[:, :, 0, 1], wt[:, :, 1, 1]], axis=-1)
    wde = jnp.stack(
        [ilv_a.reshape(NUM_R, 4, 32), ilv_b.reshape(NUM_R, 4, 32)], axis=2
    ).reshape(NUM_R, 2 * H_DIM)
    w32 = jax.lax.bitcast_convert_type(
        wde.reshape(NUM_R, H_DIM, 2), jnp.int32)

    r = np.arange(CHUNK, dtype=np.int32)
    didx_host = jnp.asarray((r // 5) * 8 + (r % 5), dtype=jnp.int32)

    parts, degs = _sc_aggregate(h_perm, src, dst, et, w32, didx_host)
    parts = parts[:, :N_NODES]
    degs = degs[:, :N_NODES]

    pm = np.zeros((H_DIM, H_DIM), dtype=np.float32)
    for j in range(HALF):
        pm[j, 2 * j] = 1.0
        pm[HALF + j, 2 * j + 1] = 1.0
    perm_mat = jnp.asarray(pm)

    return _tc_finish(parts, degs, ln_gamma.reshape(1, H_DIM),
                      ln_beta.reshape(1, H_DIM), perm_mat)


# speedup vs baseline: 6.9486x; 1.0329x over previous
import functools

import jax
import jax.numpy as jnp
import numpy as np
from jax import lax
from jax.experimental import pallas as pl
from jax.experimental.pallas import tpu as pltpu
from jax.experimental.pallas import tpu_sc as plsc

N_NODES = 10000
H_DIM = 128
NUM_R = 200
HALF = H_DIM // 2

NW = 32
CHUNK = 80
N_PAD = 10240
ROWS_PER_TILE = N_PAD // 16
DEG_ROWS = 128


def _sc_aggregate(h_perm, src, dst, etypes, w32, didx_host):
    E = src.shape[0]
    e_per_w = E // NW
    n_chunks = e_per_w // CHUNK
    mesh = plsc.VectorSubcoreMesh(core_axis_name="c", subcore_axis_name="s")

    @functools.partial(
        pl.kernel,
        mesh=mesh,
        compiler_params=pltpu.CompilerParams(needs_layout_passes=False),
        out_type=[
            jax.ShapeDtypeStruct((2, N_PAD, H_DIM), jnp.float32),
            jax.ShapeDtypeStruct((2, N_PAD, H_DIM), jnp.float32),
        ],
        scratch_types=[
            pltpu.VMEM((2, CHUNK, H_DIM), jnp.float32),
            pltpu.VMEM((2, CHUNK, H_DIM), jnp.int32),
            pltpu.VMEM((CHUNK,), jnp.int32),
            pltpu.VMEM((CHUNK,), jnp.int32),
            pltpu.VMEM((CHUNK,), jnp.int32),
            pltpu.VMEM((CHUNK,), jnp.int32),
            pltpu.VMEM((CHUNK,), jnp.int32),
            pltpu.VMEM((CHUNK,), jnp.int32),
            pltpu.VMEM((CHUNK // 2, H_DIM), jnp.int32),
            pltpu.VMEM((4, H_DIM), jnp.int32),
            pltpu.VMEM_SHARED((N_PAD, H_DIM), jnp.float32),
            pltpu.VMEM_SHARED((DEG_ROWS, H_DIM), jnp.float32),
            pltpu.SemaphoreType.DMA,
            pltpu.SemaphoreType.DMA,
            pltpu.SemaphoreType.DMA,
            pltpu.SemaphoreType.DMA,
            pltpu.SemaphoreType.DMA,
            pltpu.SemaphoreType.DMA,
        ],
    )
    def k(hh, srch, dsth, eth, wdeh, didxh, out, degout, hbr, wbr,
          srcv0, srcv1, dstv0, dstv1, etv0, etv1, degv, ohtab, agg, dagg,
          gh0, gh1, gw0, gw1, sc0, sc1):
        core = lax.axis_index("c")
        sub = lax.axis_index("s")
        wid = sub * 2 + core
        ghs = (gh0, gh1)
        gws = (gw0, gw1)
        scs = (sc0, sc1)
        srcvs = (srcv0, srcv1)
        dstvs = (dstv0, dstv1)
        etvs = (etv0, etv1)
        hb0 = hbr.at[0]

        zvec = jnp.zeros((16,), jnp.float32)
        zivec = jnp.zeros((16,), jnp.int32)
        lanes = lax.iota(jnp.int32, 16)

        def zrow(r, _):
            for j in range(H_DIM // 16):
                hbr[0, r, pl.ds(j * 16, 16)] = zvec
            return 0

        lax.fori_loop(0, CHUNK, zrow, 0)

        def zdeg(r, _):
            for j in range(H_DIM // 16):
                degv[r, pl.ds(j * 16, 16)] = zivec
            return 0

        lax.fori_loop(0, CHUNK // 2, zdeg, 0)

        def zacc(r, _):
            pltpu.async_copy(
                hb0, agg.at[pl.ds(sub * ROWS_PER_TILE + r * CHUNK, CHUNK)],
                gh0)
            return 0

        lax.fori_loop(0, ROWS_PER_TILE // CHUNK, zacc, 0)
        pltpu.sync_copy(hb0.at[pl.ds(0, 8)], dagg.at[pl.ds(sub * 8, 8)])

        def zaccw(r, _):
            pltpu.make_async_copy(
                hb0, agg.at[pl.ds(sub * ROWS_PER_TILE + r * CHUNK, CHUNK)],
                gh0).wait()
            return 0

        lax.fori_loop(0, ROWS_PER_TILE // CHUNK, zaccw, 0)

        for kk in range(32):
            ohtab[kk // 8, pl.ds((kk % 8) * 16, 16)] = jnp.where(
                lanes == (kk & 15), 1 << (16 * (kk >> 4)), 0)

        plsc.subcore_barrier()

        def load_idx(c, b):
            base = wid * e_per_w + c * CHUNK
            pltpu.sync_copy(srch.at[pl.ds(base, CHUNK)], srcvs[b])
            pltpu.sync_copy(dsth.at[pl.ds(base, CHUNK)], dstvs[b])
            pltpu.sync_copy(eth.at[pl.ds(base, CHUNK)], etvs[b])

        def start_gathers(b):
            return (pltpu.async_copy(hh.at[srcvs[b]], hbr.at[b], ghs[b]),
                    pltpu.async_copy(wdeh.at[etvs[b]], wbr.at[b], gws[b]))

        def wait_gathers(b):
            pltpu.make_async_copy(hh.at[srcvs[b]], hbr.at[b], ghs[b]).wait()
            pltpu.make_async_copy(wdeh.at[etvs[b]], wbr.at[b], gws[b]).wait()

        def compute(b):
            def edge16(i16, _):
                dstvec = dstvs[b][pl.ds(i16 * 16, 16)]
                wrowv = dstvec >> 8
                wcolv = ((dstvec >> 5) & 7) * 16
                ohidx = (dstvec & 1) * 16 + ((dstvec >> 1) & 15)
                ohrowv = ohidx >> 3
                ohcolv = (ohidx & 7) * 16
                for j in range(16):
                    i = i16 * 16 + j
                    oh = ohtab[ohrowv[j], pl.ds(ohcolv[j], 16)]
                    degv[wrowv[j], pl.ds(wcolv[j], 16)] = (
                        degv[wrowv[j], pl.ds(wcolv[j], 16)] + oh)
                    for g in range(HALF // 16):
                        he = hbr[b, i, pl.ds(g * 16, 16)]
                        ho = hbr[b, i, pl.ds(HALF + g * 16, 16)]
                        wv0 = plsc.bitcast(
                            wbr[b, i, pl.ds(g * 32, 16)], jnp.bfloat16)
                        wv1 = plsc.bitcast(
                            wbr[b, i, pl.ds(g * 32 + 16, 16)], jnp.bfloat16)
                        w00, w10 = plsc.unpack(
                            wv0, format=plsc.PackFormat.INTERLEAVED)
                        w01, w11 = plsc.unpack(
                            wv1, format=plsc.PackFormat.INTERLEAVED)
                        hbr[b, i, pl.ds(g * 16, 16)] = he * w00 + ho * w10
                        hbr[b, i, pl.ds(HALF + g * 16, 16)] = (
                            he * w01 + ho * w11)
                return 0

            lax.fori_loop(0, CHUNK // 16, edge16, 0)

        def start_scatter(b):
            return pltpu.async_copy(hbr.at[b], agg.at[dstvs[b]], scs[b],
                                    add=True)

        def wait_scatter(b):
            pltpu.make_async_copy(hbr.at[b], agg.at[dstvs[b]],
                                  scs[b]).wait()

        load_idx(0, 0)
        start_gathers(0)
        load_idx(1, 1)
        start_gathers(1)
        wait_gathers(0)
        compute(0)
        start_scatter(0)

        def step(c, b):
            wait_gathers(b)
            wait_scatter(1 - b)
            load_idx(c + 1, 1 - b)
            start_gathers(1 - b)
            compute(b)
            start_scatter(b)

        def pair(c2, _):
            c = 2 * c2 + 1
            step(c, 1)
            step(c + 1, 0)
            return 0

        lax.fori_loop(0, (n_chunks - 3) // 2, pair, 0)

        wait_gathers(1)
        wait_scatter(0)
        load_idx(n_chunks - 1, 0)
        start_gathers(0)
        compute(1)
        start_scatter(1)
        wait_gathers(0)
        wait_scatter(1)
        compute(0)
        start_scatter(0)
        wait_scatter(0)

        def dconv(r, _):
            for hrow in range(2):
                for t in range(HALF // 16):
                    v = degv[r, pl.ds(hrow * HALF + t * 16, 16)]
                    hbr[0, 2 * r + hrow, pl.ds(t * 16, 16)] = (
                        (v & 0xFFFF).astype(jnp.float32))
                    hbr[0, 2 * r + hrow, pl.ds(HALF + t * 16, 16)] = (
                        (v >> 16).astype(jnp.float32))
            return 0

        lax.fori_loop(0, CHUNK // 2, dconv, 0)
        pltpu.sync_copy(didxh, etvs[0])
        pltpu.sync_copy(hb0, dagg.at[etvs[0]], add=True)
        plsc.subcore_barrier()

        pltpu.sync_copy(dagg.at[pl.ds(sub * 8, 8)], hb0.at[pl.ds(0, 8)])

        def aggout(kk2, _):
            r0 = sub * ROWS_PER_TILE + kk2 * 32
            pltpu.async_copy(agg.at[pl.ds(r0, 32)],
                             out.at[core, pl.ds(r0, 32)], gw0)
            return 0

        lax.fori_loop(0, ROWS_PER_TILE // 32, aggout, 0)

        def outk(kk2, b2):
            s0 = 16 + b2 * 32

            @pl.when(kk2 >= 2)
            def _():
                r1 = sub * ROWS_PER_TILE + (kk2 - 2) * 32
                pltpu.make_async_copy(
                    hb0.at[pl.ds(s0, 32)],
                    degout.at[core, pl.ds(r1, 32)], scs[b2]).wait()

            q = kk2 // 4
            e0 = (kk2 - q * 4) * 16
            evec = hbr[0, q, pl.ds(e0, 16)]
            ovec = hbr[0, q, pl.ds(HALF + e0, 16)]
            for j in range(32):
                val = evec[j // 2] if j % 2 == 0 else ovec[j // 2]
                sv = jnp.full((16,), val, jnp.float32)
                for cc in range(H_DIM // 16):
                    hbr[0, s0 + j, pl.ds(cc * 16, 16)] = sv
            r0 = sub * ROWS_PER_TILE + kk2 * 32
            pltpu.async_copy(hb0.at[pl.ds(s0, 32)],
                             degout.at[core, pl.ds(r0, 32)], scs[b2])

        def outk2(kk4, _):
            outk(kk4 * 2, 0)
            outk(kk4 * 2 + 1, 1)
            return 0

        lax.fori_loop(0, ROWS_PER_TILE // 64, outk2, 0)
        for b2 in range(2):
            r1 = sub * ROWS_PER_TILE + (ROWS_PER_TILE // 32 - 2 + b2) * 32
            pltpu.make_async_copy(
                hb0.at[pl.ds(16 + b2 * 32, 32)],
                degout.at[core, pl.ds(r1, 32)], scs[b2]).wait()

        def aggoutw(kk2, _):
            r0 = sub * ROWS_PER_TILE + kk2 * 32
            pltpu.make_async_copy(agg.at[pl.ds(r0, 32)],
                                  out.at[core, pl.ds(r0, 32)], gw0).wait()
            return 0

        lax.fori_loop(0, ROWS_PER_TILE // 32, aggoutw, 0)

    return k(h_perm, src, dst, etypes, w32, didx_host)


def _tc_finish(parts, degs, gamma, beta, perm_mat):
    n_blk = 2048
    grid = (N_PAD // n_blk,)

    def body(parts_ref, deg_ref, g_ref, b_ref, p_ref, o_ref):
        x = parts_ref[0] + parts_ref[1]
        deg = deg_ref[0] + deg_ref[1]
        x = x * (1.0 / jnp.maximum(deg, 1.0))
        mean = jnp.mean(x, axis=1, keepdims=True)
        xc = x - mean
        var = jnp.mean(xc * xc, axis=1, keepdims=True)
        y = xc * lax.rsqrt(var + 1e-5)
        y = jnp.dot(y, p_ref[...], preferred_element_type=jnp.float32)
        o_ref[...] = y * g_ref[...] + b_ref[...]

    return pl.pallas_call(
        body,
        grid=grid,
        in_specs=[
            pl.BlockSpec((2, n_blk, H_DIM), lambda i: (0, i, 0)),
            pl.BlockSpec((2, n_blk, H_DIM), lambda i: (0, i, 0)),
            pl.BlockSpec((1, H_DIM), lambda i: (0, 0)),
            pl.BlockSpec((1, H_DIM), lambda i: (0, 0)),
            pl.BlockSpec((H_DIM, H_DIM), lambda i: (0, 0)),
        ],
        out_specs=pl.BlockSpec((n_blk, H_DIM), lambda i: (i, 0)),
        out_shape=jax.ShapeDtypeStruct((N_PAD, H_DIM), jnp.float32),
    )(parts, degs, gamma, beta, perm_mat)


def kernel(h, edge_index, etypes, weight, ln_gamma, ln_beta):
    src = edge_index[0].astype(jnp.int32)
    dst = edge_index[1].astype(jnp.int32)
    et = etypes.astype(jnp.int32)

    h_perm = jnp.concatenate([h[:, 0::2], h[:, 1::2]], axis=1)

    wt = weight.reshape(NUM_R, HALF, 2, 2).astype(jnp.bfloat16)
    ilv_a = jnp.stack([wt[:, :, 0, 0], wt[:, :, 1, 0]], axis=-1)
    ilv_b = jnp.stack([wt[:, :, 0, 1], wt[:, :, 1, 1]], axis=-1)
    wde = jnp.stack(
        [ilv_a.reshape(NUM_R, 4, 32), ilv_b.reshape(NUM_R, 4, 32)], axis=2
    ).reshape(NUM_R, 2 * H_DIM)
    w32 = jax.lax.bitcast_convert_type(
        wde.reshape(NUM_R, H_DIM, 2), jnp.int32)

    r = np.arange(CHUNK, dtype=np.int32)
    didx_host = jnp.asarray((r // 5) * 8 + (r % 5), dtype=jnp.int32)

    parts, degs = _sc_aggregate(h_perm, src, dst, et, w32, didx_host)

    pm = np.zeros((H_DIM, H_DIM), dtype=np.float32)
    for j in range(HALF):
        pm[j, 2 * j] = 1.0
        pm[HALF + j, 2 * j + 1] = 1.0
    perm_mat = jnp.asarray(pm)

    out = _tc_finish(parts, degs, ln_gamma.reshape(1, H_DIM),
                     ln_beta.reshape(1, H_DIM), perm_mat)
    return out[:N_NODES]


# speedup vs baseline: 7.9432x; 1.1431x over previous
import functools

import jax
import jax.numpy as jnp
import numpy as np
from jax import lax
from jax.experimental import pallas as pl
from jax.experimental.pallas import tpu as pltpu
from jax.experimental.pallas import tpu_sc as plsc

N_NODES = 10000
H_DIM = 128
NUM_R = 200
HALF = H_DIM // 2

NW = 32
CHUNK = 80
N_PAD = 10240
ROWS_PER_TILE = N_PAD // 16
DEG_ROWS = 128


def _sc_aggregate(h_perm, se, dst, w32, didx_host):
    E = dst.shape[0]
    e_per_w = E // NW
    n_chunks = e_per_w // CHUNK
    mesh = plsc.VectorSubcoreMesh(core_axis_name="c", subcore_axis_name="s")

    @functools.partial(
        pl.kernel,
        mesh=mesh,
        compiler_params=pltpu.CompilerParams(needs_layout_passes=False),
        out_type=[
            jax.ShapeDtypeStruct((2, N_PAD, H_DIM), jnp.float32),
            jax.ShapeDtypeStruct((2, N_PAD, H_DIM), jnp.float32),
        ],
        scratch_types=[
            pltpu.VMEM((2, CHUNK, H_DIM), jnp.float32),
            pltpu.VMEM((2, CHUNK, H_DIM), jnp.int32),
            pltpu.VMEM((2, CHUNK), jnp.int32),
            pltpu.VMEM((2, CHUNK), jnp.int32),
            pltpu.VMEM((CHUNK,), jnp.int32),
            pltpu.VMEM((CHUNK,), jnp.int32),
            pltpu.VMEM((CHUNK // 2, H_DIM), jnp.int32),
            pltpu.VMEM((4, H_DIM), jnp.int32),
            pltpu.VMEM_SHARED((N_PAD, H_DIM), jnp.float32),
            pltpu.VMEM_SHARED((DEG_ROWS, H_DIM), jnp.float32),
            pltpu.SemaphoreType.DMA,
            pltpu.SemaphoreType.DMA,
            pltpu.SemaphoreType.DMA,
            pltpu.SemaphoreType.DMA,
            pltpu.SemaphoreType.DMA,
            pltpu.SemaphoreType.DMA,
        ],
    )
    def k(hh, seh, dsth, wdeh, didxh, out, degout, hbr, wbr,
          sev0, sev1, dstv0, dstv1, degv, ohtab, agg, dagg,
          gh0, gh1, gw0, gw1, sc0, sc1):
        core = lax.axis_index("c")
        sub = lax.axis_index("s")
        wid = sub * 2 + core
        ghs = (gh0, gh1)
        gws = (gw0, gw1)
        scs = (sc0, sc1)
        sevs = (sev0, sev1)
        dstvs = (dstv0, dstv1)
        hb0 = hbr.at[0]

        zvec = jnp.zeros((16,), jnp.float32)
        zivec = jnp.zeros((16,), jnp.int32)
        lanes = lax.iota(jnp.int32, 16)

        def zrow(r, _):
            for j in range(H_DIM // 16):
                hbr[0, r, pl.ds(j * 16, 16)] = zvec
            return 0

        lax.fori_loop(0, CHUNK, zrow, 0)

        def zdeg(r, _):
            for j in range(H_DIM // 16):
                degv[r, pl.ds(j * 16, 16)] = zivec
            return 0

        lax.fori_loop(0, CHUNK // 2, zdeg, 0)

        def zacc(r, _):
            pltpu.async_copy(
                hb0, agg.at[pl.ds(sub * ROWS_PER_TILE + r * CHUNK, CHUNK)],
                gh0)
            return 0

        lax.fori_loop(0, ROWS_PER_TILE // CHUNK, zacc, 0)
        pltpu.sync_copy(hb0.at[pl.ds(0, 8)], dagg.at[pl.ds(sub * 8, 8)])

        def zaccw(r, _):
            pltpu.make_async_copy(
                hb0, agg.at[pl.ds(sub * ROWS_PER_TILE + r * CHUNK, CHUNK)],
                gh0).wait()
            return 0

        lax.fori_loop(0, ROWS_PER_TILE // CHUNK, zaccw, 0)

        for kk in range(32):
            ohtab[kk // 8, pl.ds((kk % 8) * 16, 16)] = jnp.where(
                lanes == (kk & 15), 1 << (16 * (kk >> 4)), 0)

        plsc.subcore_barrier()

        def load_idx(c, b):
            base = wid * e_per_w + c * CHUNK
            cp1 = pltpu.async_copy(seh.at[wid, c], sevs[b], ghs[b])
            cp2 = pltpu.async_copy(dsth.at[pl.ds(base, CHUNK)], dstvs[b],
                                   gws[b])
            cp1.wait()
            cp2.wait()

        def start_gathers(b):
            return (pltpu.async_copy(hh.at[sevs[b].at[0]], hbr.at[b],
                                     ghs[b]),
                    pltpu.async_copy(wdeh.at[sevs[b].at[1]], wbr.at[b],
                                     gws[b]))

        def wait_gathers(b):
            pltpu.make_async_copy(hh.at[sevs[b].at[0]], hbr.at[b],
                                  ghs[b]).wait()
            pltpu.make_async_copy(wdeh.at[sevs[b].at[1]], wbr.at[b],
                                  gws[b]).wait()

        def compute(b):
            def edge16(i16, _):
                dstvec = dstvs[b][pl.ds(i16 * 16, 16)]
                wrowv = dstvec >> 8
                wcolv = ((dstvec >> 5) & 7) * 16
                ohidx = (dstvec & 1) * 16 + ((dstvec >> 1) & 15)
                ohrowv = ohidx >> 3
                ohcolv = (ohidx & 7) * 16
                for j in range(16):
                    i = i16 * 16 + j
                    oh = ohtab[ohrowv[j], pl.ds(ohcolv[j], 16)]
                    degv[wrowv[j], pl.ds(wcolv[j], 16)] = (
                        degv[wrowv[j], pl.ds(wcolv[j], 16)] + oh)
                    for g in range(HALF // 16):
                        he = hbr[b, i, pl.ds(g * 16, 16)]
                        ho = hbr[b, i, pl.ds(HALF + g * 16, 16)]
                        wv0 = plsc.bitcast(
                            wbr[b, i, pl.ds(g * 32, 16)], jnp.bfloat16)
                        wv1 = plsc.bitcast(
                            wbr[b, i, pl.ds(g * 32 + 16, 16)], jnp.bfloat16)
                        w00, w10 = plsc.unpack(
                            wv0, format=plsc.PackFormat.INTERLEAVED)
                        w01, w11 = plsc.unpack(
                            wv1, format=plsc.PackFormat.INTERLEAVED)
                        hbr[b, i, pl.ds(g * 16, 16)] = he * w00 + ho * w10
                        hbr[b, i, pl.ds(HALF + g * 16, 16)] = (
                            he * w01 + ho * w11)
                return 0

            lax.fori_loop(0, CHUNK // 16, edge16, 0)

        def start_scatter(b):
            return pltpu.async_copy(hbr.at[b], agg.at[dstvs[b]], scs[b],
                                    add=True)

        def wait_scatter(b):
            pltpu.make_async_copy(hbr.at[b], agg.at[dstvs[b]],
                                  scs[b]).wait()

        load_idx(0, 0)
        start_gathers(0)
        load_idx(1, 1)
        start_gathers(1)
        wait_gathers(0)
        compute(0)
        start_scatter(0)

        def step(c, b):
            wait_gathers(b)
            wait_scatter(1 - b)
            load_idx(c + 1, 1 - b)
            start_gathers(1 - b)
            compute(b)
            start_scatter(b)

        def pair(c2, _):
            c = 2 * c2 + 1
            step(c, 1)
            step(c + 1, 0)
            return 0

        lax.fori_loop(0, (n_chunks - 3) // 2, pair, 0)

        wait_gathers(1)
        wait_scatter(0)
        load_idx(n_chunks - 1, 0)
        start_gathers(0)
        compute(1)
        start_scatter(1)
        wait_gathers(0)
        wait_scatter(1)
        compute(0)
        start_scatter(0)
        wait_scatter(0)

        def dconv(r, _):
            for hrow in range(2):
                for t in range(HALF // 16):
                    v = degv[r, pl.ds(hrow * HALF + t * 16, 16)]
                    hbr[0, 2 * r + hrow, pl.ds(t * 16, 16)] = (
                        (v & 0xFFFF).astype(jnp.float32))
                    hbr[0, 2 * r + hrow, pl.ds(HALF + t * 16, 16)] = (
                        (v >> 16).astype(jnp.float32))
            return 0

        lax.fori_loop(0, CHUNK // 2, dconv, 0)
        pltpu.sync_copy(didxh, sevs[0].at[0])
        pltpu.sync_copy(hb0, dagg.at[sevs[0].at[0]], add=True)
        plsc.subcore_barrier()

        pltpu.sync_copy(dagg.at[pl.ds(sub * 8, 8)], hb0.at[pl.ds(0, 8)])

        def aggout(kk2, _):
            r0 = sub * ROWS_PER_TILE + kk2 * 32
            pltpu.async_copy(agg.at[pl.ds(r0, 32)],
                             out.at[core, pl.ds(r0, 32)], gw0)
            return 0

        lax.fori_loop(0, ROWS_PER_TILE // 32, aggout, 0)

        def outk(kk2, b2):
            s0 = 16 + b2 * 32

            @pl.when(kk2 >= 2)
            def _():
                r1 = sub * ROWS_PER_TILE + (kk2 - 2) * 32
                pltpu.make_async_copy(
                    hb0.at[pl.ds(s0, 32)],
                    degout.at[core, pl.ds(r1, 32)], scs[b2]).wait()

            q = kk2 // 4
            e0 = (kk2 - q * 4) * 16
            evec = hbr[0, q, pl.ds(e0, 16)]
            ovec = hbr[0, q, pl.ds(HALF + e0, 16)]
            for j in range(32):
                val = evec[j // 2] if j % 2 == 0 else ovec[j // 2]
                sv = jnp.full((16,), val, jnp.float32)
                for cc in range(H_DIM // 16):
                    hbr[0, s0 + j, pl.ds(cc * 16, 16)] = sv
            r0 = sub * ROWS_PER_TILE + kk2 * 32
            pltpu.async_copy(hb0.at[pl.ds(s0, 32)],
                             degout.at[core, pl.ds(r0, 32)], scs[b2])

        def outk2(kk4, _):
            outk(kk4 * 2, 0)
            outk(kk4 * 2 + 1, 1)
            return 0

        lax.fori_loop(0, ROWS_PER_TILE // 64, outk2, 0)
        for b2 in range(2):
            r1 = sub * ROWS_PER_TILE + (ROWS_PER_TILE // 32 - 2 + b2) * 32
            pltpu.make_async_copy(
                hb0.at[pl.ds(16 + b2 * 32, 32)],
                degout.at[core, pl.ds(r1, 32)], scs[b2]).wait()

        def aggoutw(kk2, _):
            r0 = sub * ROWS_PER_TILE + kk2 * 32
            pltpu.make_async_copy(agg.at[pl.ds(r0, 32)],
                                  out.at[core, pl.ds(r0, 32)], gw0).wait()
            return 0

        lax.fori_loop(0, ROWS_PER_TILE // 32, aggoutw, 0)

    return k(h_perm, se, dst, w32, didx_host)


def _tc_finish(parts, degs, gamma, beta, perm_mat):
    n_blk = 2048
    grid = (N_PAD // n_blk,)

    def body(parts_ref, deg_ref, g_ref, b_ref, p_ref, o_ref):
        x = parts_ref[0] + parts_ref[1]
        deg = deg_ref[0] + deg_ref[1]
        x = x * (1.0 / jnp.maximum(deg, 1.0))
        mean = jnp.mean(x, axis=1, keepdims=True)
        xc = x - mean
        var = jnp.mean(xc * xc, axis=1, keepdims=True)
        y = xc * lax.rsqrt(var + 1e-5)
        y = jnp.dot(y, p_ref[...], preferred_element_type=jnp.float32)
        o_ref[...] = y * g_ref[...] + b_ref[...]

    return pl.pallas_call(
        body,
        grid=grid,
        in_specs=[
            pl.BlockSpec((2, n_blk, H_DIM), lambda i: (0, i, 0)),
            pl.BlockSpec((2, n_blk, H_DIM), lambda i: (0, i, 0)),
            pl.BlockSpec((1, H_DIM), lambda i: (0, 0)),
            pl.BlockSpec((1, H_DIM), lambda i: (0, 0)),
            pl.BlockSpec((H_DIM, H_DIM), lambda i: (0, 0)),
        ],
        out_specs=pl.BlockSpec((n_blk, H_DIM), lambda i: (i, 0)),
        out_shape=jax.ShapeDtypeStruct((N_PAD, H_DIM), jnp.float32),
    )(parts, degs, gamma, beta, perm_mat)


def kernel(h, edge_index, etypes, weight, ln_gamma, ln_beta):
    src = edge_index[0].astype(jnp.int32)
    dst = edge_index[1].astype(jnp.int32)
    et = etypes.astype(jnp.int32)

    h_perm = jnp.concatenate([h[:, 0::2], h[:, 1::2]], axis=1)

    wt = weight.reshape(NUM_R, HALF, 2, 2).astype(jnp.bfloat16)
    ilv_a = jnp.stack([wt[:, :, 0, 0], wt[:, :, 1, 0]], axis=-1)
    ilv_b = jnp.stack([wt[:, :, 0, 1], wt[:, :, 1, 1]], axis=-1)
    wde = jnp.stack(
        [ilv_a.reshape(NUM_R, 4, 32), ilv_b.reshape(NUM_R, 4, 32)], axis=2
    ).reshape(NUM_R, 2 * H_DIM)
    w32 = jax.lax.bitcast_convert_type(
        wde.reshape(NUM_R, H_DIM, 2), jnp.int32)

    r = np.arange(CHUNK, dtype=np.int32)
    didx_host = jnp.asarray((r // 5) * 8 + (r % 5), dtype=jnp.int32)

    n_chunks = src.shape[0] // (NW * CHUNK)
    se = jnp.stack([src, et], axis=0).reshape(2, NW, n_chunks, CHUNK)
    se = se.transpose(1, 2, 0, 3)
    parts, degs = _sc_aggregate(h_perm, se, dst, w32, didx_host)

    pm = np.zeros((H_DIM, H_DIM), dtype=np.float32)
    for j in range(HALF):
        pm[j, 2 * j] = 1.0
        pm[HALF + j, 2 * j + 1] = 1.0
    perm_mat = jnp.asarray(pm)

    out = _tc_finish(parts, degs, ln_gamma.reshape(1, H_DIM),
                     ln_beta.reshape(1, H_DIM), perm_mat)
    return out[:N_NODES]


# speedup vs baseline: 7.9588x; 1.0020x over previous
import functools

import jax
import jax.numpy as jnp
import numpy as np
from jax import lax
from jax.experimental import pallas as pl
from jax.experimental.pallas import tpu as pltpu
from jax.experimental.pallas import tpu_sc as plsc

N_NODES = 10000
H_DIM = 128
NUM_R = 200
HALF = H_DIM // 2

NW = 32
CHUNK = 80
N_PAD = 10240
ROWS_PER_TILE = N_PAD // 16
DEG_ROWS = 128


def _sc_aggregate(h_pk, se, dst, w32, didx_host):
    E = dst.shape[0]
    e_per_w = E // NW
    n_chunks = e_per_w // CHUNK
    mesh = plsc.VectorSubcoreMesh(core_axis_name="c", subcore_axis_name="s")

    @functools.partial(
        pl.kernel,
        mesh=mesh,
        compiler_params=pltpu.CompilerParams(needs_layout_passes=False),
        out_type=[
            jax.ShapeDtypeStruct((2, N_PAD, H_DIM), jnp.float32),
            jax.ShapeDtypeStruct((2, N_PAD, H_DIM), jnp.float32),
        ],
        scratch_types=[
            pltpu.VMEM((2, CHUNK, H_DIM), jnp.float32),
            pltpu.VMEM((2, CHUNK, H_DIM), jnp.float32),
            pltpu.VMEM((2, CHUNK), jnp.int32),
            pltpu.VMEM((2, CHUNK), jnp.int32),
            pltpu.VMEM((CHUNK,), jnp.int32),
            pltpu.VMEM((CHUNK,), jnp.int32),
            pltpu.VMEM((CHUNK // 2, H_DIM), jnp.int32),
            pltpu.VMEM((4, H_DIM), jnp.int32),
            pltpu.VMEM_SHARED((N_PAD, H_DIM), jnp.float32),
            pltpu.VMEM_SHARED((DEG_ROWS, H_DIM), jnp.float32),
            pltpu.SemaphoreType.DMA,
            pltpu.SemaphoreType.DMA,
            pltpu.SemaphoreType.DMA,
            pltpu.SemaphoreType.DMA,
            pltpu.SemaphoreType.DMA,
            pltpu.SemaphoreType.DMA,
        ],
    )
    def k(hh, seh, dsth, wdeh, didxh, out, degout, hbr, wbr,
          sev0, sev1, dstv0, dstv1, degv, ohtab, agg, dagg,
          gh0, gh1, gw0, gw1, sc0, sc1):
        core = lax.axis_index("c")
        sub = lax.axis_index("s")
        wid = sub * 2 + core
        ghs = (gh0, gh1)
        gws = (gw0, gw1)
        scs = (sc0, sc1)
        sevs = (sev0, sev1)
        dstvs = (dstv0, dstv1)
        wb0 = wbr.at[0]

        zvec = jnp.zeros((16,), jnp.float32)
        zivec = jnp.zeros((16,), jnp.int32)
        lanes = lax.iota(jnp.int32, 16)

        def zrow(r, _):
            for j in range(H_DIM // 16):
                wbr[0, r, pl.ds(j * 16, 16)] = zvec
            return 0

        lax.fori_loop(0, CHUNK, zrow, 0)

        def zdeg(r, _):
            for j in range(H_DIM // 16):
                degv[r, pl.ds(j * 16, 16)] = zivec
            return 0

        lax.fori_loop(0, CHUNK // 2, zdeg, 0)

        def zacc(r, _):
            pltpu.async_copy(
                wb0, agg.at[pl.ds(sub * ROWS_PER_TILE + r * CHUNK, CHUNK)],
                gh0)
            return 0

        lax.fori_loop(0, ROWS_PER_TILE // CHUNK, zacc, 0)
        pltpu.sync_copy(wb0.at[pl.ds(0, 8)], dagg.at[pl.ds(sub * 8, 8)])

        def zaccw(r, _):
            pltpu.make_async_copy(
                wb0, agg.at[pl.ds(sub * ROWS_PER_TILE + r * CHUNK, CHUNK)],
                gh0).wait()
            return 0

        lax.fori_loop(0, ROWS_PER_TILE // CHUNK, zaccw, 0)

        for kk in range(32):
            ohtab[kk // 8, pl.ds((kk % 8) * 16, 16)] = jnp.where(
                lanes == (kk & 15), 1 << (16 * (kk >> 4)), 0)

        plsc.subcore_barrier()

        def load_idx(c, b):
            base = wid * e_per_w + c * CHUNK
            cp1 = pltpu.async_copy(seh.at[wid, c], sevs[b], ghs[b])
            cp2 = pltpu.async_copy(dsth.at[pl.ds(base, CHUNK)], dstvs[b],
                                   gws[b])
            cp1.wait()
            cp2.wait()

        def start_gathers(b):
            return (pltpu.async_copy(hh.at[sevs[b].at[0]], hbr.at[b],
                                     ghs[b]),
                    pltpu.async_copy(wdeh.at[sevs[b].at[1]], wbr.at[b],
                                     gws[b]))

        def wait_gathers(b):
            pltpu.make_async_copy(hh.at[sevs[b].at[0]], hbr.at[b],
                                  ghs[b]).wait()
            pltpu.make_async_copy(wdeh.at[sevs[b].at[1]], wbr.at[b],
                                  gws[b]).wait()

        def compute(b):
            def edge16(i16, _):
                dstvec = dstvs[b][pl.ds(i16 * 16, 16)]
                wrowv = dstvec >> 8
                wcolv = ((dstvec >> 5) & 7) * 16
                ohidx = (dstvec & 1) * 16 + ((dstvec >> 1) & 15)
                ohrowv = ohidx >> 3
                ohcolv = (ohidx & 7) * 16
                for j in range(16):
                    i = i16 * 16 + j
                    oh = ohtab[ohrowv[j], pl.ds(ohcolv[j], 16)]
                    degv[wrowv[j], pl.ds(wcolv[j], 16)] = (
                        degv[wrowv[j], pl.ds(wcolv[j], 16)] + oh)
                    mes = []
                    mos = []
                    for g in range(HALF // 16):
                        he = hbr[b, i, pl.ds(g * 16, 16)]
                        ho = hbr[b, i, pl.ds(HALF + g * 16, 16)]
                        wv0 = plsc.bitcast(
                            wbr[b, i, pl.ds(g * 32, 16)], jnp.bfloat16)
                        wv1 = plsc.bitcast(
                            wbr[b, i, pl.ds(g * 32 + 16, 16)], jnp.bfloat16)
                        w00, w10 = plsc.unpack(
                            wv0, format=plsc.PackFormat.INTERLEAVED)
                        w01, w11 = plsc.unpack(
                            wv1, format=plsc.PackFormat.INTERLEAVED)
                        mes.append(he * w00 + ho * w10)
                        mos.append(he * w01 + ho * w11)
                    for g in range(HALF // 16):
                        wbr[b, i, pl.ds(g * 16, 16)] = mes[g]
                        wbr[b, i, pl.ds(HALF + g * 16, 16)] = mos[g]
                return 0

            lax.fori_loop(0, CHUNK // 16, edge16, 0)

        def start_scatter(b):
            return pltpu.async_copy(wbr.at[b], agg.at[dstvs[b]], scs[b],
                                    add=True)

        def wait_scatter(b):
            pltpu.make_async_copy(wbr.at[b], agg.at[dstvs[b]],
                                  scs[b]).wait()

        load_idx(0, 0)
        start_gathers(0)
        load_idx(1, 1)
        start_gathers(1)
        wait_gathers(0)
        compute(0)
        start_scatter(0)

        def step(c, b):
            wait_gathers(b)
            wait_scatter(1 - b)
            load_idx(c + 1, 1 - b)
            start_gathers(1 - b)
            compute(b)
            start_scatter(b)

        def pair(c2, _):
            c = 2 * c2 + 1
            step(c, 1)
            step(c + 1, 0)
            return 0

        lax.fori_loop(0, (n_chunks - 3) // 2, pair, 0)

        wait_gathers(1)
        wait_scatter(0)
        load_idx(n_chunks - 1, 0)
        start_gathers(0)
        compute(1)
        start_scatter(1)
        wait_gathers(0)
        wait_scatter(1)
        compute(0)
        start_scatter(0)
        wait_scatter(0)

        def dconv(r, _):
            for hrow in range(2):
                for t in range(HALF // 16):
                    v = degv[r, pl.ds(hrow * HALF + t * 16, 16)]
                    wbr[0, 2 * r + hrow, pl.ds(t * 16, 16)] = (
                        (v & 0xFFFF).astype(jnp.float32))
                    wbr[0, 2 * r + hrow, pl.ds(HALF + t * 16, 16)] = (
                        (v >> 16).astype(jnp.float32))
            return 0

        lax.fori_loop(0, CHUNK // 2, dconv, 0)
        pltpu.sync_copy(didxh, sevs[0].at[0])
        pltpu.sync_copy(wb0, dagg.at[sevs[0].at[0]], add=True)
        plsc.subcore_barrier()

        pltpu.sync_copy(dagg.at[pl.ds(sub * 8, 8)], wb0.at[pl.ds(0, 8)])

        def aggout(kk2, _):
            r0 = sub * ROWS_PER_TILE + kk2 * 32
            pltpu.async_copy(agg.at[pl.ds(r0, 32)],
                             out.at[core, pl.ds(r0, 32)], gw0)
            return 0

        lax.fori_loop(0, ROWS_PER_TILE // 32, aggout, 0)

        def outk(kk2, b2):
            s0 = 16 + b2 * 32

            @pl.when(kk2 >= 2)
            def _():
                r1 = sub * ROWS_PER_TILE + (kk2 - 2) * 32
                pltpu.make_async_copy(
                    wb0.at[pl.ds(s0, 32)],
                    degout.at[core, pl.ds(r1, 32)], scs[b2]).wait()

            q = kk2 // 4
            e0 = (kk2 - q * 4) * 16
            evec = wbr[0, q, pl.ds(e0, 16)]
            ovec = wbr[0, q, pl.ds(HALF + e0, 16)]
            for j in range(32):
                val = evec[j // 2] if j % 2 == 0 else ovec[j // 2]
                sv = jnp.full((16,), val, jnp.float32)
                for cc in range(H_DIM // 16):
                    wbr[0, s0 + j, pl.ds(cc * 16, 16)] = sv
            r0 = sub * ROWS_PER_TILE + kk2 * 32
            pltpu.async_copy(wb0.at[pl.ds(s0, 32)],
                             degout.at[core, pl.ds(r0, 32)], scs[b2])

        def outk2(kk4, _):
            outk(kk4 * 2, 0)
            outk(kk4 * 2 + 1, 1)
            return 0

        lax.fori_loop(0, ROWS_PER_TILE // 64, outk2, 0)
        for b2 in range(2):
            r1 = sub * ROWS_PER_TILE + (ROWS_PER_TILE // 32 - 2 + b2) * 32
            pltpu.make_async_copy(
                wb0.at[pl.ds(16 + b2 * 32, 32)],
                degout.at[core, pl.ds(r1, 32)], scs[b2]).wait()

        def aggoutw(kk2, _):
            r0 = sub * ROWS_PER_TILE + kk2 * 32
            pltpu.make_async_copy(agg.at[pl.ds(r0, 32)],
                                  out.at[core, pl.ds(r0, 32)], gw0).wait()
            return 0

        lax.fori_loop(0, ROWS_PER_TILE // 32, aggoutw, 0)

    return k(h_pk, se, dst, w32, didx_host)


def _tc_finish(parts, degs, gamma, beta, perm_mat):
    n_blk = 2048
    grid = (N_PAD // n_blk,)

    def body(parts_ref, deg_ref, g_ref, b_ref, p_ref, o_ref):
        x = parts_ref[0] + parts_ref[1]
        deg = deg_ref[0] + deg_ref[1]
        x = x * (1.0 / jnp.maximum(deg, 1.0))
        mean = jnp.mean(x, axis=1, keepdims=True)
        xc = x - mean
        var = jnp.mean(xc * xc, axis=1, keepdims=True)
        y = xc * lax.rsqrt(var + 1e-5)
        y = jnp.dot(y, p_ref[...], preferred_element_type=jnp.float32)
        o_ref[...] = y * g_ref[...] + b_ref[...]

    return pl.pallas_call(
        body,
        grid=grid,
        in_specs=[
            pl.BlockSpec((2, n_blk, H_DIM), lambda i: (0, i, 0)),
            pl.BlockSpec((2, n_blk, H_DIM), lambda i: (0, i, 0)),
            pl.BlockSpec((1, H_DIM), lambda i: (0, 0)),
            pl.BlockSpec((1, H_DIM), lambda i: (0, 0)),
            pl.BlockSpec((H_DIM, H_DIM), lambda i: (0, 0)),
        ],
        out_specs=pl.BlockSpec((n_blk, H_DIM), lambda i: (i, 0)),
        out_shape=jax.ShapeDtypeStruct((N_PAD, H_DIM), jnp.float32),
    )(parts, degs, gamma, beta, perm_mat)


def kernel(h, edge_index, etypes, weight, ln_gamma, ln_beta):
    src = edge_index[0].astype(jnp.int32)
    dst = edge_index[1].astype(jnp.int32)
    et = etypes.astype(jnp.int32)

    h_pk = jnp.concatenate([h[:, 0::2], h[:, 1::2]], axis=1)

    wt = weight.reshape(NUM_R, HALF, 2, 2).astype(jnp.bfloat16)
    ilv_a = jnp.stack([wt[:, :, 0, 0], wt[:, :, 1, 0]], axis=-1)
    ilv_b = jnp.stack([wt[:, :, 0, 1], wt[:, :, 1, 1]], axis=-1)
    wde = jnp.stack(
        [ilv_a.reshape(NUM_R, 4, 32), ilv_b.reshape(NUM_R, 4, 32)], axis=2
    ).reshape(NUM_R, 2 * H_DIM)
    w32 = jax.lax.bitcast_convert_type(
        wde.reshape(NUM_R, H_DIM, 2), jnp.float32)

    r = np.arange(CHUNK, dtype=np.int32)
    didx_host = jnp.asarray((r // 5) * 8 + (r % 5), dtype=jnp.int32)

    n_chunks = src.shape[0] // (NW * CHUNK)
    se = jnp.stack([src, et], axis=0).reshape(2, NW, n_chunks, CHUNK)
    se = se.transpose(1, 2, 0, 3)
    parts, degs = _sc_aggregate(h_pk, se, dst, w32, didx_host)

    pm = np.zeros((H_DIM, H_DIM), dtype=np.float32)
    for j in range(HALF):
        pm[j, 2 * j] = 1.0
        pm[HALF + j, 2 * j + 1] = 1.0
    perm_mat = jnp.asarray(pm)

    out = _tc_finish(parts, degs, ln_gamma.reshape(1, H_DIM),
                     ln_beta.reshape(1, H_DIM), perm_mat)
    return out[:N_NODES]


# speedup vs baseline: 7.9819x; 1.0029x over previous
import functools

import jax
import jax.numpy as jnp
import numpy as np
from jax import lax
from jax.experimental import pallas as pl
from jax.experimental.pallas import tpu as pltpu
from jax.experimental.pallas import tpu_sc as plsc

N_NODES = 10000
H_DIM = 128
NUM_R = 200
HALF = H_DIM // 2

NW = 32
CHUNK = 80
N_PAD = 10240
ROWS_PER_TILE = N_PAD // 16
DEG_ROWS = 128


def _sc_aggregate(h_pk, se, dst, w32, didx_host):
    E = dst.shape[0]
    e_per_w = E // NW
    n_chunks = e_per_w // CHUNK
    mesh = plsc.VectorSubcoreMesh(core_axis_name="c", subcore_axis_name="s")

    @functools.partial(
        pl.kernel,
        mesh=mesh,
        compiler_params=pltpu.CompilerParams(needs_layout_passes=False),
        out_type=[
            jax.ShapeDtypeStruct((2, N_PAD, H_DIM), jnp.float32),
            jax.ShapeDtypeStruct((2, N_PAD, H_DIM), jnp.float32),
        ],
        scratch_types=[
            pltpu.VMEM((2, CHUNK, H_DIM), jnp.float32),
            pltpu.VMEM((2, CHUNK, H_DIM), jnp.float32),
            pltpu.VMEM((2, CHUNK), jnp.int32),
            pltpu.VMEM((2, CHUNK), jnp.int32),
            pltpu.VMEM((CHUNK,), jnp.int32),
            pltpu.VMEM((CHUNK,), jnp.int32),
            pltpu.VMEM((CHUNK // 2 * H_DIM,), jnp.int32),
            pltpu.VMEM((512,), jnp.int32),
            pltpu.VMEM_SHARED((N_PAD, H_DIM), jnp.float32),
            pltpu.VMEM_SHARED((DEG_ROWS, H_DIM), jnp.float32),
            pltpu.SemaphoreType.DMA,
            pltpu.SemaphoreType.DMA,
            pltpu.SemaphoreType.DMA,
            pltpu.SemaphoreType.DMA,
            pltpu.SemaphoreType.DMA,
            pltpu.SemaphoreType.DMA,
        ],
    )
    def k(hh, seh, dsth, wdeh, didxh, out, degout, hbr, wbr,
          sev0, sev1, dstv0, dstv1, degv, ohtab, agg, dagg,
          gh0, gh1, gw0, gw1, sc0, sc1):
        core = lax.axis_index("c")
        sub = lax.axis_index("s")
        wid = sub * 2 + core
        ghs = (gh0, gh1)
        gws = (gw0, gw1)
        scs = (sc0, sc1)
        sevs = (sev0, sev1)
        dstvs = (dstv0, dstv1)
        wb0 = wbr.at[0]

        zvec = jnp.zeros((16,), jnp.float32)
        zivec = jnp.zeros((16,), jnp.int32)
        lanes = lax.iota(jnp.int32, 16)

        def zrow(r, _):
            for j in range(H_DIM // 16):
                wbr[0, r, pl.ds(j * 16, 16)] = zvec
            return 0

        lax.fori_loop(0, CHUNK, zrow, 0)

        def zdeg(r, _):
            degv[pl.ds(r * 16, 16)] = zivec
            return 0

        lax.fori_loop(0, CHUNK // 2 * H_DIM // 16, zdeg, 0)

        def zacc(r, _):
            pltpu.async_copy(
                wb0, agg.at[pl.ds(sub * ROWS_PER_TILE + r * CHUNK, CHUNK)],
                gh0)
            return 0

        lax.fori_loop(0, ROWS_PER_TILE // CHUNK, zacc, 0)
        pltpu.sync_copy(wb0.at[pl.ds(0, 8)], dagg.at[pl.ds(sub * 8, 8)])

        def zaccw(r, _):
            pltpu.make_async_copy(
                wb0, agg.at[pl.ds(sub * ROWS_PER_TILE + r * CHUNK, CHUNK)],
                gh0).wait()
            return 0

        lax.fori_loop(0, ROWS_PER_TILE // CHUNK, zaccw, 0)

        for kk in range(32):
            ohtab[pl.ds(kk * 16, 16)] = jnp.where(
                lanes == (kk & 15), 1 << (16 * (kk >> 4)), 0)

        plsc.subcore_barrier()

        def load_idx(c, b):
            base = wid * e_per_w + c * CHUNK
            cp1 = pltpu.async_copy(seh.at[wid, c], sevs[b], ghs[b])
            cp2 = pltpu.async_copy(dsth.at[pl.ds(base, CHUNK)], dstvs[b],
                                   gws[b])
            cp1.wait()
            cp2.wait()

        def start_gathers(b):
            return (pltpu.async_copy(hh.at[sevs[b].at[0]], hbr.at[b],
                                     ghs[b]),
                    pltpu.async_copy(wdeh.at[sevs[b].at[1]], wbr.at[b],
                                     gws[b]))

        def wait_gathers(b):
            pltpu.make_async_copy(hh.at[sevs[b].at[0]], hbr.at[b],
                                  ghs[b]).wait()
            pltpu.make_async_copy(wdeh.at[sevs[b].at[1]], wbr.at[b],
                                  gws[b]).wait()

        def compute(b):
            def edge16(i16, _):
                dstvec = dstvs[b][pl.ds(i16 * 16, 16)]
                daddrv = (dstvec >> 1) & 0xFFF0
                ohaddrv = ((dstvec & 1) * 16 + ((dstvec >> 1) & 15)) * 16
                for j in range(16):
                    i = i16 * 16 + j
                    da = daddrv[j]
                    oh = ohtab[pl.ds(ohaddrv[j], 16)]
                    degv[pl.ds(da, 16)] = degv[pl.ds(da, 16)] + oh
                    mes = []
                    mos = []
                    for g in range(HALF // 16):
                        he = hbr[b, i, pl.ds(g * 16, 16)]
                        ho = hbr[b, i, pl.ds(HALF + g * 16, 16)]
                        wv0 = plsc.bitcast(
                            wbr[b, i, pl.ds(g * 32, 16)], jnp.bfloat16)
                        wv1 = plsc.bitcast(
                            wbr[b, i, pl.ds(g * 32 + 16, 16)], jnp.bfloat16)
                        w00, w10 = plsc.unpack(
                            wv0, format=plsc.PackFormat.INTERLEAVED)
                        w01, w11 = plsc.unpack(
                            wv1, format=plsc.PackFormat.INTERLEAVED)
                        mes.append(he * w00 + ho * w10)
                        mos.append(he * w01 + ho * w11)
                    for g in range(HALF // 16):
                        wbr[b, i, pl.ds(g * 16, 16)] = mes[g]
                        wbr[b, i, pl.ds(HALF + g * 16, 16)] = mos[g]
                return 0

            lax.fori_loop(0, CHUNK // 16, edge16, 0)

        def start_scatter(b):
            return pltpu.async_copy(wbr.at[b], agg.at[dstvs[b]], scs[b],
                                    add=True)

        def wait_scatter(b):
            pltpu.make_async_copy(wbr.at[b], agg.at[dstvs[b]],
                                  scs[b]).wait()

        load_idx(0, 0)
        start_gathers(0)
        load_idx(1, 1)
        start_gathers(1)
        wait_gathers(0)
        compute(0)
        start_scatter(0)

        def step(c, b):
            wait_gathers(b)
            wait_scatter(1 - b)
            load_idx(c + 1, 1 - b)
            start_gathers(1 - b)
            compute(b)
            start_scatter(b)

        def pair(c2, _):
            c = 2 * c2 + 1
            step(c, 1)
            step(c + 1, 0)
            return 0

        lax.fori_loop(0, (n_chunks - 3) // 2, pair, 0)

        wait_gathers(1)
        wait_scatter(0)
        load_idx(n_chunks - 1, 0)
        start_gathers(0)
        compute(1)
        start_scatter(1)
        wait_gathers(0)
        wait_scatter(1)
        compute(0)
        start_scatter(0)
        wait_scatter(0)

        def dconv(r, _):
            for hrow in range(2):
                for t in range(HALF // 16):
                    v = degv[pl.ds(r * H_DIM + hrow * HALF + t * 16, 16)]
                    wbr[0, 2 * r + hrow, pl.ds(t * 16, 16)] = (
                        (v & 0xFFFF).astype(jnp.float32))
                    wbr[0, 2 * r + hrow, pl.ds(HALF + t * 16, 16)] = (
                        (v >> 16).astype(jnp.float32))
            return 0

        lax.fori_loop(0, CHUNK // 2, dconv, 0)
        pltpu.sync_copy(didxh, sevs[0].at[0])
        pltpu.sync_copy(wb0, dagg.at[sevs[0].at[0]], add=True)
        plsc.subcore_barrier()

        pltpu.sync_copy(dagg.at[pl.ds(sub * 8, 8)], wb0.at[pl.ds(0, 8)])

        def aggout(kk2, _):
            r0 = sub * ROWS_PER_TILE + kk2 * 32
            pltpu.async_copy(agg.at[pl.ds(r0, 32)],
                             out.at[core, pl.ds(r0, 32)], gw0)
            return 0

        lax.fori_loop(0, ROWS_PER_TILE // 32, aggout, 0)

        def outk(kk2, b2):
            s0 = 16 + b2 * 32

            @pl.when(kk2 >= 2)
            def _():
                r1 = sub * ROWS_PER_TILE + (kk2 - 2) * 32
                pltpu.make_async_copy(
                    wb0.at[pl.ds(s0, 32)],
                    degout.at[core, pl.ds(r1, 32)], scs[b2]).wait()

            q = kk2 // 4
            e0 = (kk2 - q * 4) * 16
            evec = wbr[0, q, pl.ds(e0, 16)]
            ovec = wbr[0, q, pl.ds(HALF + e0, 16)]
            for j in range(32):
                val = evec[j // 2] if j % 2 == 0 else ovec[j // 2]
                sv = jnp.full((16,), val, jnp.float32)
                for cc in range(H_DIM // 16):
                    wbr[0, s0 + j, pl.ds(cc * 16, 16)] = sv
            r0 = sub * ROWS_PER_TILE + kk2 * 32
            pltpu.async_copy(wb0.at[pl.ds(s0, 32)],
                             degout.at[core, pl.ds(r0, 32)], scs[b2])

        def outk2(kk4, _):
            outk(kk4 * 2, 0)
            outk(kk4 * 2 + 1, 1)
            return 0

        lax.fori_loop(0, ROWS_PER_TILE // 64, outk2, 0)
        for b2 in range(2):
            r1 = sub * ROWS_PER_TILE + (ROWS_PER_TILE // 32 - 2 + b2) * 32
            pltpu.make_async_copy(
                wb0.at[pl.ds(16 + b2 * 32, 32)],
                degout.at[core, pl.ds(r1, 32)], scs[b2]).wait()

        def aggoutw(kk2, _):
            r0 = sub * ROWS_PER_TILE + kk2 * 32
            pltpu.make_async_copy(agg.at[pl.ds(r0, 32)],
                                  out.at[core, pl.ds(r0, 32)], gw0).wait()
            return 0

        lax.fori_loop(0, ROWS_PER_TILE // 32, aggoutw, 0)

    return k(h_pk, se, dst, w32, didx_host)


def _tc_finish(parts, degs, gamma, beta, perm_mat):
    n_blk = 2048
    grid = (N_PAD // n_blk,)

    def body(parts_ref, deg_ref, g_ref, b_ref, p_ref, o_ref):
        x = parts_ref[0] + parts_ref[1]
        deg = deg_ref[0] + deg_ref[1]
        x = x * (1.0 / jnp.maximum(deg, 1.0))
        mean = jnp.mean(x, axis=1, keepdims=True)
        xc = x - mean
        var = jnp.mean(xc * xc, axis=1, keepdims=True)
        y = xc * lax.rsqrt(var + 1e-5)
        y = jnp.dot(y, p_ref[...], preferred_element_type=jnp.float32)
        o_ref[...] = y * g_ref[...] + b_ref[...]

    return pl.pallas_call(
        body,
        grid=grid,
        in_specs=[
            pl.BlockSpec((2, n_blk, H_DIM), lambda i: (0, i, 0)),
            pl.BlockSpec((2, n_blk, H_DIM), lambda i: (0, i, 0)),
            pl.BlockSpec((1, H_DIM), lambda i: (0, 0)),
            pl.BlockSpec((1, H_DIM), lambda i: (0, 0)),
            pl.BlockSpec((H_DIM, H_DIM), lambda i: (0, 0)),
        ],
        out_specs=pl.BlockSpec((n_blk, H_DIM), lambda i: (i, 0)),
        out_shape=jax.ShapeDtypeStruct((N_PAD, H_DIM), jnp.float32),
    )(parts, degs, gamma, beta, perm_mat)


def kernel(h, edge_index, etypes, weight, ln_gamma, ln_beta):
    src = edge_index[0].astype(jnp.int32)
    dst = edge_index[1].astype(jnp.int32)
    et = etypes.astype(jnp.int32)

    h_pk = jnp.concatenate([h[:, 0::2], h[:, 1::2]], axis=1)

    wt = weight.reshape(NUM_R, HALF, 2, 2).astype(jnp.bfloat16)
    ilv_a = jnp.stack([wt[:, :, 0, 0], wt[:, :, 1, 0]], axis=-1)
    ilv_b = jnp.stack([wt[:, :, 0, 1], wt[:, :, 1, 1]], axis=-1)
    wde = jnp.stack(
        [ilv_a.reshape(NUM_R, 4, 32), ilv_b.reshape(NUM_R, 4, 32)], axis=2
    ).reshape(NUM_R, 2 * H_DIM)
    w32 = jax.lax.bitcast_convert_type(
        wde.reshape(NUM_R, H_DIM, 2), jnp.float32)

    r = np.arange(CHUNK, dtype=np.int32)
    didx_host = jnp.asarray((r // 5) * 8 + (r % 5), dtype=jnp.int32)

    n_chunks = src.shape[0] // (NW * CHUNK)
    se = jnp.stack([src, et], axis=0).reshape(2, NW, n_chunks, CHUNK)
    se = se.transpose(1, 2, 0, 3)
    parts, degs = _sc_aggregate(h_pk, se, dst, w32, didx_host)

    pm = np.zeros((H_DIM, H_DIM), dtype=np.float32)
    for j in range(HALF):
        pm[j, 2 * j] = 1.0
        pm[HALF + j, 2 * j + 1] = 1.0
    perm_mat = jnp.asarray(pm)

    out = _tc_finish(parts, degs, ln_gamma.reshape(1, H_DIM),
                     ln_beta.reshape(1, H_DIM), perm_mat)
    return out[:N_NODES]


# speedup vs baseline: 7.9856x; 1.0005x over previous
import functools

import jax
import jax.numpy as jnp
import numpy as np
from jax import lax
from jax.experimental import pallas as pl
from jax.experimental.pallas import tpu as pltpu
from jax.experimental.pallas import tpu_sc as plsc

N_NODES = 10000
H_DIM = 128
NUM_R = 200
HALF = H_DIM // 2

NW = 32
CHUNK = 80
N_PAD = 10240
ROWS_PER_TILE = N_PAD // 16
DEG_ROWS = 128


def _sc_aggregate(h_pk, se, dst, w32, didx_host):
    E = dst.shape[0]
    e_per_w = E // NW
    n_chunks = e_per_w // CHUNK
    mesh = plsc.VectorSubcoreMesh(core_axis_name="c", subcore_axis_name="s")

    @functools.partial(
        pl.kernel,
        mesh=mesh,
        compiler_params=pltpu.CompilerParams(needs_layout_passes=False),
        out_type=[
            jax.ShapeDtypeStruct((2, N_PAD, H_DIM), jnp.float32),
            jax.ShapeDtypeStruct((2, N_PAD, H_DIM), jnp.float32),
        ],
        scratch_types=[
            pltpu.VMEM((2, CHUNK, H_DIM), jnp.float32),
            pltpu.VMEM((2, CHUNK, H_DIM), jnp.float32),
            pltpu.VMEM((2, CHUNK), jnp.int32),
            pltpu.VMEM((2, CHUNK), jnp.int32),
            pltpu.VMEM((CHUNK,), jnp.int32),
            pltpu.VMEM((CHUNK,), jnp.int32),
            pltpu.VMEM((CHUNK // 2 * H_DIM,), jnp.int32),
            pltpu.VMEM((512,), jnp.int32),
            pltpu.VMEM_SHARED((N_PAD, H_DIM), jnp.float32),
            pltpu.VMEM_SHARED((DEG_ROWS, H_DIM), jnp.float32),
            pltpu.SemaphoreType.DMA,
            pltpu.SemaphoreType.DMA,
            pltpu.SemaphoreType.DMA,
            pltpu.SemaphoreType.DMA,
            pltpu.SemaphoreType.DMA,
            pltpu.SemaphoreType.DMA,
        ],
    )
    def k(hh, seh, dsth, wdeh, didxh, out, degout, hbr, wbr,
          sev0, sev1, dstv0, dstv1, degv, ohtab, agg, dagg,
          gh0, gh1, gw0, gw1, sc0, sc1):
        core = lax.axis_index("c")
        sub = lax.axis_index("s")
        wid = sub * 2 + core
        ghs = (gh0, gh1)
        gws = (gw0, gw1)
        scs = (sc0, sc1)
        sevs = (sev0, sev1)
        dstvs = (dstv0, dstv1)
        wb0 = wbr.at[0]

        zvec = jnp.zeros((16,), jnp.float32)
        zivec = jnp.zeros((16,), jnp.int32)
        lanes = lax.iota(jnp.int32, 16)

        def zrow(r, _):
            for j in range(H_DIM // 16):
                wbr[0, r, pl.ds(j * 16, 16)] = zvec
            return 0

        lax.fori_loop(0, CHUNK, zrow, 0)

        def zdeg(r, _):
            degv[pl.ds(r * 16, 16)] = zivec
            return 0

        lax.fori_loop(0, CHUNK // 2 * H_DIM // 16, zdeg, 0)

        def zacc(r, _):
            pltpu.async_copy(
                wb0, agg.at[pl.ds(sub * ROWS_PER_TILE + r * CHUNK, CHUNK)],
                gh0)
            return 0

        lax.fori_loop(0, ROWS_PER_TILE // CHUNK, zacc, 0)
        pltpu.sync_copy(wb0.at[pl.ds(0, 8)], dagg.at[pl.ds(sub * 8, 8)])

        def zaccw(r, _):
            pltpu.make_async_copy(
                wb0, agg.at[pl.ds(sub * ROWS_PER_TILE + r * CHUNK, CHUNK)],
                gh0).wait()
            return 0

        lax.fori_loop(0, ROWS_PER_TILE // CHUNK, zaccw, 0)

        for kk in range(32):
            ohtab[pl.ds(kk * 16, 16)] = jnp.where(
                lanes == (kk & 15), 1 << (16 * (kk >> 4)), 0)

        plsc.subcore_barrier()

        def load_idx_start(c, b):
            base = wid * e_per_w + c * CHUNK
            return (pltpu.async_copy(seh.at[wid, c], sevs[b], ghs[b]),
                    pltpu.async_copy(dsth.at[pl.ds(base, CHUNK)], dstvs[b],
                                     gws[b]))

        def load_idx(c, b):
            cp1, cp2 = load_idx_start(c, b)
            cp1.wait()
            cp2.wait()

        def start_gathers(b):
            return (pltpu.async_copy(hh.at[sevs[b].at[0]], hbr.at[b],
                                     ghs[b]),
                    pltpu.async_copy(wdeh.at[sevs[b].at[1]], wbr.at[b],
                                     gws[b]))

        def wait_gathers(b):
            pltpu.make_async_copy(hh.at[sevs[b].at[0]], hbr.at[b],
                                  ghs[b]).wait()
            pltpu.make_async_copy(wdeh.at[sevs[b].at[1]], wbr.at[b],
                                  gws[b]).wait()

        def compute(b):
            def edge16(i16, _):
                dstvec = dstvs[b][pl.ds(i16 * 16, 16)]
                daddrv = (dstvec >> 1) & 0xFFF0
                ohaddrv = ((dstvec & 1) * 16 + ((dstvec >> 1) & 15)) * 16
                for j in range(16):
                    i = i16 * 16 + j
                    da = daddrv[j]
                    oh = ohtab[pl.ds(ohaddrv[j], 16)]
                    degv[pl.ds(da, 16)] = degv[pl.ds(da, 16)] + oh
                    mes = []
                    mos = []
                    for g in range(HALF // 16):
                        he = hbr[b, i, pl.ds(g * 16, 16)]
                        ho = hbr[b, i, pl.ds(HALF + g * 16, 16)]
                        wv0 = plsc.bitcast(
                            wbr[b, i, pl.ds(g * 32, 16)], jnp.bfloat16)
                        wv1 = plsc.bitcast(
                            wbr[b, i, pl.ds(g * 32 + 16, 16)], jnp.bfloat16)
                        w00, w10 = plsc.unpack(
                            wv0, format=plsc.PackFormat.INTERLEAVED)
                        w01, w11 = plsc.unpack(
                            wv1, format=plsc.PackFormat.INTERLEAVED)
                        mes.append(he * w00 + ho * w10)
                        mos.append(he * w01 + ho * w11)
                    for g in range(HALF // 16):
                        wbr[b, i, pl.ds(g * 16, 16)] = mes[g]
                        wbr[b, i, pl.ds(HALF + g * 16, 16)] = mos[g]
                return 0

            lax.fori_loop(0, CHUNK // 16, edge16, 0)

        def start_scatter(b):
            return pltpu.async_copy(wbr.at[b], agg.at[dstvs[b]], scs[b],
                                    add=True)

        def wait_scatter(b):
            pltpu.make_async_copy(wbr.at[b], agg.at[dstvs[b]],
                                  scs[b]).wait()

        load_idx(0, 0)
        start_gathers(0)
        load_idx(1, 1)
        start_gathers(1)
        wait_gathers(0)
        compute(0)
        start_scatter(0)

        def step(c, b):
            wait_scatter(1 - b)
            cp1, cp2 = load_idx_start(c + 1, 1 - b)
            wait_gathers(b)
            cp1.wait()
            cp2.wait()
            start_gathers(1 - b)
            compute(b)
            start_scatter(b)

        def pair(c2, _):
            c = 2 * c2 + 1
            step(c, 1)
            step(c + 1, 0)
            return 0

        lax.fori_loop(0, (n_chunks - 3) // 2, pair, 0)

        wait_gathers(1)
        wait_scatter(0)
        load_idx(n_chunks - 1, 0)
        start_gathers(0)
        compute(1)
        start_scatter(1)
        wait_gathers(0)
        wait_scatter(1)
        compute(0)
        start_scatter(0)
        wait_scatter(0)

        def dconv(r, _):
            for hrow in range(2):
                for t in range(HALF // 16):
                    v = degv[pl.ds(r * H_DIM + hrow * HALF + t * 16, 16)]
                    wbr[0, 2 * r + hrow, pl.ds(t * 16, 16)] = (
                        (v & 0xFFFF).astype(jnp.float32))
                    wbr[0, 2 * r + hrow, pl.ds(HALF + t * 16, 16)] = (
                        (v >> 16).astype(jnp.float32))
            return 0

        lax.fori_loop(0, CHUNK // 2, dconv, 0)
        pltpu.sync_copy(didxh, sevs[0].at[0])
        pltpu.sync_copy(wb0, dagg.at[sevs[0].at[0]], add=True)
        plsc.subcore_barrier()

        pltpu.sync_copy(dagg.at[pl.ds(sub * 8, 8)], wb0.at[pl.ds(0, 8)])

        def aggout(kk2, _):
            r0 = sub * ROWS_PER_TILE + kk2 * 32
            pltpu.async_copy(agg.at[pl.ds(r0, 32)],
                             out.at[core, pl.ds(r0, 32)], gw0)
            return 0

        lax.fori_loop(0, ROWS_PER_TILE // 32, aggout, 0)

        def outk(kk2, b2):
            s0 = 16 + b2 * 32

            @pl.when(kk2 >= 2)
            def _():
                r1 = sub * ROWS_PER_TILE + (kk2 - 2) * 32
                pltpu.make_async_copy(
                    wb0.at[pl.ds(s0, 32)],
                    degout.at[core, pl.ds(r1, 32)], scs[b2]).wait()

            q = kk2 // 4
            e0 = (kk2 - q * 4) * 16
            evec = wbr[0, q, pl.ds(e0, 16)]
            ovec = wbr[0, q, pl.ds(HALF + e0, 16)]
            for j in range(32):
                val = evec[j // 2] if j % 2 == 0 else ovec[j // 2]
                sv = jnp.full((16,), val, jnp.float32)
                for cc in range(H_DIM // 16):
                    wbr[0, s0 + j, pl.ds(cc * 16, 16)] = sv
            r0 = sub * ROWS_PER_TILE + kk2 * 32
            pltpu.async_copy(wb0.at[pl.ds(s0, 32)],
                             degout.at[core, pl.ds(r0, 32)], scs[b2])

        def outk2(kk4, _):
            outk(kk4 * 2, 0)
            outk(kk4 * 2 + 1, 1)
            return 0

        lax.fori_loop(0, ROWS_PER_TILE // 64, outk2, 0)
        for b2 in range(2):
            r1 = sub * ROWS_PER_TILE + (ROWS_PER_TILE // 32 - 2 + b2) * 32
            pltpu.make_async_copy(
                wb0.at[pl.ds(16 + b2 * 32, 32)],
                degout.at[core, pl.ds(r1, 32)], scs[b2]).wait()

        def aggoutw(kk2, _):
            r0 = sub * ROWS_PER_TILE + kk2 * 32
            pltpu.make_async_copy(agg.at[pl.ds(r0, 32)],
                                  out.at[core, pl.ds(r0, 32)], gw0).wait()
            return 0

        lax.fori_loop(0, ROWS_PER_TILE // 32, aggoutw, 0)

    return k(h_pk, se, dst, w32, didx_host)


def _tc_finish(parts, degs, gamma, beta, perm_mat):
    n_blk = 2048
    grid = (N_PAD // n_blk,)

    def body(parts_ref, deg_ref, g_ref, b_ref, p_ref, o_ref):
        x = parts_ref[0] + parts_ref[1]
        deg = deg_ref[0] + deg_ref[1]
        x = x * (1.0 / jnp.maximum(deg, 1.0))
        mean = jnp.mean(x, axis=1, keepdims=True)
        xc = x - mean
        var = jnp.mean(xc * xc, axis=1, keepdims=True)
        y = xc * lax.rsqrt(var + 1e-5)
        y = jnp.dot(y, p_ref[...], preferred_element_type=jnp.float32)
        o_ref[...] = y * g_ref[...] + b_ref[...]

    return pl.pallas_call(
        body,
        grid=grid,
        in_specs=[
            pl.BlockSpec((2, n_blk, H_DIM), lambda i: (0, i, 0)),
            pl.BlockSpec((2, n_blk, H_DIM), lambda i: (0, i, 0)),
            pl.BlockSpec((1, H_DIM), lambda i: (0, 0)),
            pl.BlockSpec((1, H_DIM), lambda i: (0, 0)),
            pl.BlockSpec((H_DIM, H_DIM), lambda i: (0, 0)),
        ],
        out_specs=pl.BlockSpec((n_blk, H_DIM), lambda i: (i, 0)),
        out_shape=jax.ShapeDtypeStruct((N_PAD, H_DIM), jnp.float32),
    )(parts, degs, gamma, beta, perm_mat)


def kernel(h, edge_index, etypes, weight, ln_gamma, ln_beta):
    src = edge_index[0].astype(jnp.int32)
    dst = edge_index[1].astype(jnp.int32)
    et = etypes.astype(jnp.int32)

    h_pk = jnp.concatenate([h[:, 0::2], h[:, 1::2]], axis=1)

    wt = weight.reshape(NUM_R, HALF, 2, 2).astype(jnp.bfloat16)
    ilv_a = jnp.stack([wt[:, :, 0, 0], wt[:, :, 1, 0]], axis=-1)
    ilv_b = jnp.stack([wt[:, :, 0, 1], wt[:, :, 1, 1]], axis=-1)
    wde = jnp.stack(
        [ilv_a.reshape(NUM_R, 4, 32), ilv_b.reshape(NUM_R, 4, 32)], axis=2
    ).reshape(NUM_R, 2 * H_DIM)
    w32 = jax.lax.bitcast_convert_type(
        wde.reshape(NUM_R, H_DIM, 2), jnp.float32)

    r = np.arange(CHUNK, dtype=np.int32)
    didx_host = jnp.asarray((r // 5) * 8 + (r % 5), dtype=jnp.int32)

    n_chunks = src.shape[0] // (NW * CHUNK)
    se = jnp.stack([src, et], axis=0).reshape(2, NW, n_chunks, CHUNK)
    se = se.transpose(1, 2, 0, 3)
    parts, degs = _sc_aggregate(h_pk, se, dst, w32, didx_host)

    pm = np.zeros((H_DIM, H_DIM), dtype=np.float32)
    for j in range(HALF):
        pm[j, 2 * j] = 1.0
        pm[HALF + j, 2 * j + 1] = 1.0
    perm_mat = jnp.asarray(pm)

    out = _tc_finish(parts, degs, ln_gamma.reshape(1, H_DIM),
                     ln_beta.reshape(1, H_DIM), perm_mat)
    return out[:N_NODES]
